# baseline (speedup 1.0000x reference)
import functools

import jax
import jax.numpy as jnp
from jax import lax
from jax.experimental import pallas as pl
from jax.experimental.pallas import tpu as pltpu


def _agg_kernel(x_ref, o_ref, *, t, inv_t, band0_end, band2_start):
    bblk, _, c = x_ref.shape
    g = c // t
    n = bblk * t

    xf = x_ref[...].reshape(n, c)
    xb = xf.astype(jnp.bfloat16)

    jj = lax.broadcasted_iota(jnp.int32, (c, c), 0)
    kk = lax.broadcasted_iota(jnp.int32, (c, c), 1)
    base = (kk % g) * t
    lo = base + jnp.where(kk >= band2_start, 1, 0)
    hi = base + t - jnp.where(kk < band0_end, 1, 0)
    qc = ((jj >= lo) & (jj < hi)).astype(jnp.bfloat16)

    cand = jnp.dot(xb, qc, preferred_element_type=jnp.float32)

    row = lax.broadcasted_iota(jnp.int32, (n, c), 0)
    col = lax.broadcasted_iota(jnp.int32, (n, c), 1)
    maskb = ((row % t) == (col // g)).astype(jnp.bfloat16)

    z = xb + maskb * cand.astype(jnp.bfloat16)

    rb = lax.broadcasted_iota(jnp.int32, (bblk, n), 0)
    rr = lax.broadcasted_iota(jnp.int32, (bblk, n), 1)
    s = (rr // t == rb).astype(jnp.bfloat16)

    o_ref[...] = (jnp.dot(s, z, preferred_element_type=jnp.float32)
                  * inv_t).astype(o_ref.dtype)


def kernel(x):
    b, t, c = x.shape
    assert c % t == 0
    bblk = min(b, 256)
    params = pltpu.CompilerParams(
        dimension_semantics=("parallel",),
        vmem_limit_bytes=52 << 20,
    )
    return pl.pallas_call(
        functools.partial(
            _agg_kernel, t=t, inv_t=1.0 / t,
            band0_end=c // 4, band2_start=c + (-c // 4)),
        out_shape=jax.ShapeDtypeStruct((b, c), x.dtype),
        grid=(pl.cdiv(b, bblk),),
        in_specs=[pl.BlockSpec((bblk, t, c), lambda i: (i, 0, 0))],
        out_specs=pl.BlockSpec((bblk, c), lambda i: (i, 0)),
        compiler_params=params,
    )(x)
